# baseline (speedup 1.0000x reference)
import jax
import jax.numpy as jnp
from jax import lax
from jax.experimental import pallas as pl
from jax.experimental.pallas import tpu as pltpu
from jax.experimental.pallas import tpu_sc as plsc

N_CLASSES = 1000
CTX_LEN = 77
D_MODEL = 512

NC, NS = 2, 16
NW = NC * NS
B = N_CLASSES * CTX_LEN
K = 56
NCHUNK = 22
HPW = K * NCHUNK
BPW = 2 * HPW
STRIDE = 2408


def _gather_body(table_hbm, idx_hbm, out_hbm, idx_v,
                 a0, a1, b0, b1, ga0, ga1, gb0, gb1, wa, wb):
    c = lax.axis_index("c")
    s = lax.axis_index("s")
    wid = s * NC + c
    base = pl.multiple_of(jnp.minimum(wid * STRIDE, B - BPW), 8)
    pltpu.sync_copy(idx_hbm.at[pl.ds(base, BPW)], idx_v)

    abufs, bbufs = (a0, a1), (b0, b1)
    agsems, bgsems = (ga0, ga1), (gb0, gb1)

    def gather(off, buf, sem):
        pltpu.async_copy(table_hbm.at[idx_v.at[pl.ds(off, K)]], buf, sem)

    gather(0, abufs[0], agsems[0])
    gather(HPW, bbufs[0], bgsems[0])

    def body(g, carry):
        for b in range(2):
            i = g * 2 + b
            nb = 1 - b
            pltpu.make_async_copy(table_hbm.at[idx_v.at[pl.ds(0, K)]],
                                  abufs[b], agsems[b]).wait()
            pltpu.make_async_copy(table_hbm.at[idx_v.at[pl.ds(0, K)]],
                                  bbufs[b], bgsems[b]).wait()

            @pl.when(i + 1 < NCHUNK)
            def _():
                off = (i + 1) * K
                gather(off, abufs[nb], agsems[nb])
                gather(HPW + off, bbufs[nb], bgsems[nb])

            ha = pltpu.make_async_copy(
                abufs[b], out_hbm.at[pl.ds(base + i * K, K)], wa)
            ha.start()
            hb = pltpu.make_async_copy(
                bbufs[b], out_hbm.at[pl.ds(base + HPW + i * K, K)], wb)
            hb.start()
            ha.wait()
            hb.wait()
        return carry

    lax.fori_loop(0, NCHUNK // 2, body, 0)


def _sc_gather(table, idx_pad):
    mesh = plsc.VectorSubcoreMesh(core_axis_name="c", subcore_axis_name="s")
    f = pl.kernel(
        _gather_body,
        out_type=jax.ShapeDtypeStruct((B, D_MODEL), jnp.float32),
        mesh=mesh,
        scratch_types=[
            pltpu.VMEM((BPW,), jnp.int32),
            pltpu.VMEM((K, D_MODEL), jnp.float32),
            pltpu.VMEM((K, D_MODEL), jnp.float32),
            pltpu.VMEM((K, D_MODEL), jnp.float32),
            pltpu.VMEM((K, D_MODEL), jnp.float32),
            pltpu.SemaphoreType.DMA,
            pltpu.SemaphoreType.DMA,
            pltpu.SemaphoreType.DMA,
            pltpu.SemaphoreType.DMA,
            pltpu.SemaphoreType.DMA,
            pltpu.SemaphoreType.DMA,
        ],
        name="sc_embedding_gather",
    )
    return f(table, idx_pad)


def _argmax_body(ids_ref, out_ref):
    ids = ids_ref[...]
    iota = lax.broadcasted_iota(jnp.int32, ids.shape, 1)
    m = jnp.max(ids, axis=1, keepdims=True)
    cand = jnp.where(ids == m, iota, CTX_LEN)
    out_ref[...] = jnp.min(cand, axis=1, keepdims=True)


def _tc_argmax(prompt):
    return pl.pallas_call(
        _argmax_body,
        out_shape=jax.ShapeDtypeStruct((N_CLASSES, 1), jnp.int32),
    )(prompt)


def kernel(prompt, table):
    idx = jnp.swapaxes(prompt, 0, 1).reshape(-1)
    rows = _sc_gather(table, idx)
    embedding = jnp.swapaxes(rows.reshape(CTX_LEN, N_CLASSES, D_MODEL), 0, 1)
    eos = _tc_argmax(prompt).reshape(N_CLASSES)
    return (embedding, eos)
